# Initial kernel scaffold; baseline (speedup 1.0000x reference)
import jax
import jax.numpy as jnp
from jax.experimental import pallas as pl


def kernel(x, edge_index, cov, W_ab, b_ab, emb, W_dist, b_dist, eps0, c0_W1, c0_b1, c0_W2, c0_b2, eps1, c1_W1, c1_b1, c1_W2, c1_b2, head_W1, head_b1, head_W2, head_b2):
    raise NotImplementedError("write your pallas kernel here")


# speedup vs baseline: 2.9987x; 2.9987x over previous
import functools

import jax
import jax.numpy as jnp
from jax import lax
from jax.experimental import pallas as pl
from jax.experimental.pallas import tpu as pltpu
from jax.experimental.pallas import tpu_sc as plsc

V = 100000
N = 100000
E = 1600000
HID = 48
EMB = 24
L = 16
NCH = 3
NC = 2
NS = 16
NW = NC * NS

BN = 1000
GT = N // BN

EB = 128
NB_FULL = N // EB
TAIL_N = N - NB_FULL * EB
BPT = (NB_FULL + 1 + NW - 1) // NW

EPT = E // NW
NB_E = EPT // EB
TAIL_E = EPT - NB_E * EB
ROWS_T = 6256
ROWS_LAST = N - (NS - 1) * ROWS_T


def _tc_pre_body(emb_ref, xa_ref, xd_ref, wab_ref, bab_ref, wdist_ref,
                 bdist_ref, w1a_ref, w1t_ref, w1d_ref,
                 ta_ref, tb_ref, tc_ref, ba_ref, bb_ref, bc_ref):
    e = emb_ref[...]
    T = jnp.dot(e, w1t_ref[...], preferred_element_type=jnp.float32)
    ab = xa_ref[...]
    dist = xd_ref[...]
    a_emb = jax.nn.relu(ab * wab_ref[...] + bab_ref[...])
    d_emb = jax.nn.relu(dist * wdist_ref[...] + bdist_ref[...])
    base = (jnp.dot(a_emb, w1a_ref[...], preferred_element_type=jnp.float32)
            + jnp.dot(d_emb, w1d_ref[...], preferred_element_type=jnp.float32))
    ta_ref[...] = T[:, 0:L]
    tb_ref[...] = T[:, L:2 * L]
    tc_ref[...] = T[:, 2 * L:3 * L]
    ba_ref[...] = base[:, 0:L]
    bb_ref[...] = base[:, L:2 * L]
    bc_ref[...] = base[:, 2 * L:3 * L]


_tc_pre = pl.pallas_call(
    _tc_pre_body,
    grid=(GT,),
    in_specs=[
        pl.BlockSpec((BN, EMB), lambda i: (i, 0)),
        pl.BlockSpec((BN, 1), lambda i: (i, 0)),
        pl.BlockSpec((BN, 1), lambda i: (i, 0)),
        pl.BlockSpec((1, HID), lambda i: (0, 0)),
        pl.BlockSpec((1, HID), lambda i: (0, 0)),
        pl.BlockSpec((1, 32), lambda i: (0, 0)),
        pl.BlockSpec((1, 32), lambda i: (0, 0)),
        pl.BlockSpec((HID, HID), lambda i: (0, 0)),
        pl.BlockSpec((EMB, HID), lambda i: (0, 0)),
        pl.BlockSpec((32, HID), lambda i: (0, 0)),
    ],
    out_specs=[pl.BlockSpec((BN, L), lambda i: (i, 0))] * 6,
    out_shape=[jax.ShapeDtypeStruct((N, L), jnp.float32)] * 6,
)


def _sc_embed_body(tid_ref, ta_ref, tb_ref, tc_ref, b0_ref, b1_ref, b2_ref,
                   qa_ref, qb_ref, qc_ref,
                   idxf, trowf, browf, idxt, trowt, browt, sem):
    cid = lax.axis_index("c")
    sid = lax.axis_index("s")
    wid = cid * NS + sid
    Ts = (ta_ref, tb_ref, tc_ref)
    Bs = (b0_ref, b1_ref, b2_ref)
    Qs = (qa_ref, qb_ref, qc_ref)

    @pl.loop(0, BPT)
    def _batches(k):
        g = wid * BPT + k

        @pl.when(g < NB_FULL)
        def _full():
            r0 = g * EB
            pltpu.sync_copy(tid_ref.at[pl.ds(r0, EB)], idxf)
            for c in range(NCH):
                pltpu.async_copy(Ts[c].at[idxf], trowf, sem).wait()
                pltpu.sync_copy(Bs[c].at[pl.ds(r0, EB)], browf)
                for j in range(EB):
                    trowf[j, :] = trowf[j, :] + browf[j, :]
                pltpu.sync_copy(trowf, Qs[c].at[pl.ds(r0, EB)])

        @pl.when(g == NB_FULL)
        def _tail():
            r0 = NB_FULL * EB
            pltpu.sync_copy(tid_ref.at[pl.ds(r0, TAIL_N)], idxt)
            for c in range(NCH):
                pltpu.async_copy(Ts[c].at[idxt], trowt, sem).wait()
                pltpu.sync_copy(Bs[c].at[pl.ds(r0, TAIL_N)], browt)
                for j in range(TAIL_N):
                    trowt[j, :] = trowt[j, :] + browt[j, :]
                pltpu.sync_copy(trowt, Qs[c].at[pl.ds(r0, TAIL_N)])


@functools.cache
def _get_sc_embed():
    return pl.kernel(
        _sc_embed_body,
        out_type=[jax.ShapeDtypeStruct((N, L), jnp.float32)] * 3,
        mesh=plsc.VectorSubcoreMesh(core_axis_name="c", subcore_axis_name="s"),
        compiler_params=pltpu.CompilerParams(use_tc_tiling_on_sc=False),
        scratch_types=[
            pltpu.VMEM((EB,), jnp.int32),
            pltpu.VMEM((EB, L), jnp.float32),
            pltpu.VMEM((EB, L), jnp.float32),
            pltpu.VMEM((TAIL_N,), jnp.int32),
            pltpu.VMEM((TAIL_N, L), jnp.float32),
            pltpu.VMEM((TAIL_N, L), jnp.float32),
            pltpu.SemaphoreType.DMA,
        ],
    )


def _sc_scatter_body(qa_ref, qb_ref, qc_ref, src_ref, dst_ref, zero_ref,
                     aa_ref, ab_ref, ac_ref,
                     acc, srcf, dstf, rowf, srct, dstt, rowt, sem):
    cid = lax.axis_index("c")
    sid = lax.axis_index("s")
    Qs = (qa_ref, qb_ref, qc_ref)
    As = (aa_ref, ab_ref, ac_ref)
    e_base = (cid * NS + sid) * EPT
    r0 = sid * ROWS_T

    for c in range(NCH):
        @pl.when(sid < NS - 1)
        def _zero():
            pltpu.sync_copy(zero_ref.at[pl.ds(0, ROWS_T)],
                            acc.at[pl.ds(r0, ROWS_T)])

        @pl.when(sid == NS - 1)
        def _zero_last():
            pltpu.sync_copy(zero_ref.at[pl.ds(0, ROWS_LAST)],
                            acc.at[pl.ds(r0, ROWS_LAST)])

        plsc.subcore_barrier()

        @pl.loop(0, NB_E)
        def _edge_batch(k):
            e0 = e_base + k * EB
            pltpu.sync_copy(src_ref.at[pl.ds(e0, EB)], srcf)
            pltpu.sync_copy(dst_ref.at[pl.ds(e0, EB)], dstf)
            pltpu.async_copy(Qs[c].at[srcf], rowf, sem).wait()
            pltpu.sync_copy(rowf, acc.at[dstf], add=True)

        et = e_base + NB_E * EB
        pltpu.sync_copy(src_ref.at[pl.ds(et, TAIL_E)], srct)
        pltpu.sync_copy(dst_ref.at[pl.ds(et, TAIL_E)], dstt)
        pltpu.async_copy(Qs[c].at[srct], rowt, sem).wait()
        pltpu.sync_copy(rowt, acc.at[dstt], add=True)

        plsc.subcore_barrier()

        o0 = cid * N + r0

        @pl.when(sid < NS - 1)
        def _dump():
            pltpu.sync_copy(acc.at[pl.ds(r0, ROWS_T)],
                            As[c].at[pl.ds(o0, ROWS_T)])

        @pl.when(sid == NS - 1)
        def _dump_last():
            pltpu.sync_copy(acc.at[pl.ds(r0, ROWS_LAST)],
                            As[c].at[pl.ds(o0, ROWS_LAST)])

        plsc.subcore_barrier()


@functools.cache
def _get_sc_scatter():
    return pl.kernel(
        _sc_scatter_body,
        out_type=[jax.ShapeDtypeStruct((NC * N, L), jnp.float32)] * 3,
        mesh=plsc.VectorSubcoreMesh(core_axis_name="c", subcore_axis_name="s"),
        compiler_params=pltpu.CompilerParams(use_tc_tiling_on_sc=False),
        scratch_types=[
            pltpu.VMEM_SHARED((N, L), jnp.float32),
            pltpu.VMEM((EB,), jnp.int32),
            pltpu.VMEM((EB,), jnp.int32),
            pltpu.VMEM((EB, L), jnp.float32),
            pltpu.VMEM((TAIL_E,), jnp.int32),
            pltpu.VMEM((TAIL_E,), jnp.int32),
            pltpu.VMEM((TAIL_E, L), jnp.float32),
            pltpu.SemaphoreType.DMA,
        ],
    )


def _tc_mlp1_body(qa_ref, qb_ref, qc_ref, aa_ref, ab_ref, ac_ref,
                  eps_ref, b1_ref, w2_ref, b2_ref, wn_ref,
                  oa_ref, ob_ref, oc_ref):
    q = jnp.concatenate([qa_ref[...], qb_ref[...], qc_ref[...]], axis=1)
    aa = aa_ref[...]
    ab = ab_ref[...]
    ac = ac_ref[...]
    ag = jnp.concatenate([aa[0] + aa[1], ab[0] + ab[1], ac[0] + ac[1]],
                         axis=1)
    u = jax.nn.relu((1.0 + eps_ref[0, 0]) * q + ag + b1_ref[...])
    h = jax.nn.relu(
        jnp.dot(u, w2_ref[...], preferred_element_type=jnp.float32)
        + b2_ref[...])
    qn = jnp.dot(h, wn_ref[...], preferred_element_type=jnp.float32)
    oa_ref[...] = qn[:, 0:L]
    ob_ref[...] = qn[:, L:2 * L]
    oc_ref[...] = qn[:, 2 * L:3 * L]


_tc_mlp1 = pl.pallas_call(
    _tc_mlp1_body,
    grid=(GT,),
    in_specs=[
        pl.BlockSpec((BN, L), lambda i: (i, 0)),
        pl.BlockSpec((BN, L), lambda i: (i, 0)),
        pl.BlockSpec((BN, L), lambda i: (i, 0)),
        pl.BlockSpec((NC, BN, L), lambda i: (0, i, 0)),
        pl.BlockSpec((NC, BN, L), lambda i: (0, i, 0)),
        pl.BlockSpec((NC, BN, L), lambda i: (0, i, 0)),
        pl.BlockSpec((1, 1), lambda i: (0, 0)),
        pl.BlockSpec((1, HID), lambda i: (0, 0)),
        pl.BlockSpec((HID, HID), lambda i: (0, 0)),
        pl.BlockSpec((1, HID), lambda i: (0, 0)),
        pl.BlockSpec((HID, HID), lambda i: (0, 0)),
    ],
    out_specs=[pl.BlockSpec((BN, L), lambda i: (i, 0))] * 3,
    out_shape=[jax.ShapeDtypeStruct((N, L), jnp.float32)] * 3,
)


def _tc_mlp2_body(qa_ref, qb_ref, qc_ref, aa_ref, ab_ref, ac_ref,
                  eps_ref, b1_ref, w2_ref, b2_ref,
                  cov_ref, hw1_ref, hb1_ref, hw2_ref, hb2_ref,
                  out_ref, acc_ref):
    i = pl.program_id(0)
    q = jnp.concatenate([qa_ref[...], qb_ref[...], qc_ref[...]], axis=1)
    aa = aa_ref[...]
    ab = ab_ref[...]
    ac = ac_ref[...]
    ag = jnp.concatenate([aa[0] + aa[1], ab[0] + ab[1], ac[0] + ac[1]],
                         axis=1)
    u = jax.nn.relu((1.0 + eps_ref[0, 0]) * q + ag + b1_ref[...])
    h = jax.nn.relu(
        jnp.dot(u, w2_ref[...], preferred_element_type=jnp.float32)
        + b2_ref[...])
    bsum = jnp.sum(h, axis=0, keepdims=True)

    @pl.when(i == 0)
    def _init():
        acc_ref[...] = bsum

    @pl.when(i > 0)
    def _accum():
        acc_ref[...] = acc_ref[...] + bsum

    @pl.when(i == GT - 1)
    def _head():
        gm = acc_ref[...] * (1.0 / N)
        comb = jnp.concatenate([gm, cov_ref[...]], axis=1)
        t = jax.nn.relu(
            jnp.dot(comb.astype(jnp.bfloat16),
                    hw1_ref[...].astype(jnp.bfloat16),
                    preferred_element_type=jnp.float32)
            + hb1_ref[...])
        out_ref[...] = (
            jnp.sum(t * hw2_ref[...], axis=1, keepdims=True)
            + hb2_ref[...])


_tc_mlp2 = pl.pallas_call(
    _tc_mlp2_body,
    grid=(GT,),
    in_specs=[
        pl.BlockSpec((BN, L), lambda i: (i, 0)),
        pl.BlockSpec((BN, L), lambda i: (i, 0)),
        pl.BlockSpec((BN, L), lambda i: (i, 0)),
        pl.BlockSpec((NC, BN, L), lambda i: (0, i, 0)),
        pl.BlockSpec((NC, BN, L), lambda i: (0, i, 0)),
        pl.BlockSpec((NC, BN, L), lambda i: (0, i, 0)),
        pl.BlockSpec((1, 1), lambda i: (0, 0)),
        pl.BlockSpec((1, HID), lambda i: (0, 0)),
        pl.BlockSpec((HID, HID), lambda i: (0, 0)),
        pl.BlockSpec((1, HID), lambda i: (0, 0)),
        pl.BlockSpec((1, 4), lambda i: (0, 0)),
        pl.BlockSpec((HID + 4, 64), lambda i: (0, 0)),
        pl.BlockSpec((1, 64), lambda i: (0, 0)),
        pl.BlockSpec((1, 64), lambda i: (0, 0)),
        pl.BlockSpec((1, 1), lambda i: (0, 0)),
    ],
    out_specs=pl.BlockSpec((1, 1), lambda i: (0, 0)),
    out_shape=jax.ShapeDtypeStruct((1, 1), jnp.float32),
    scratch_shapes=[pltpu.VMEM((1, HID), jnp.float32)],
)


def kernel(x, edge_index, cov, W_ab, b_ab, emb, W_dist, b_dist, eps0,
           c0_W1, c0_b1, c0_W2, c0_b2, eps1, c1_W1, c1_b1, c1_W2, c1_b2,
           head_W1, head_b1, head_W2, head_b2):
    tid = jnp.clip(x[:, 1].astype(jnp.int32), 0, V - 1)
    xa = x[:, 0:1]
    xd = x[:, 2:3]
    src = edge_index[0]
    dst = edge_index[1]
    zeros = jnp.zeros((ROWS_T, L), jnp.float32)

    w1a = c0_W1[0:HID]
    w1t = c0_W1[HID:HID + EMB]
    w1d = c0_W1[HID + EMB:]

    ta, tb, tc, ba, bb, bc = _tc_pre(
        emb, xa, xd, W_ab, b_ab.reshape(1, -1), W_dist,
        b_dist.reshape(1, -1), w1a, w1t, w1d)

    qa, qb, qc = _get_sc_embed()(tid, ta, tb, tc, ba, bb, bc)

    aa, ab, ac = _get_sc_scatter()(qa, qb, qc, src, dst, zeros)
    aa = aa.reshape(NC, N, L)
    ab = ab.reshape(NC, N, L)
    ac = ac.reshape(NC, N, L)

    q1a, q1b, q1c = _tc_mlp1(
        qa, qb, qc, aa, ab, ac, eps0.reshape(1, 1), c0_b1.reshape(1, -1),
        c0_W2, c0_b2.reshape(1, -1), c1_W1)

    ba1, bb1, bc1 = _get_sc_scatter()(q1a, q1b, q1c, src, dst, zeros)
    ba1 = ba1.reshape(NC, N, L)
    bb1 = bb1.reshape(NC, N, L)
    bc1 = bc1.reshape(NC, N, L)

    out = _tc_mlp2(
        q1a, q1b, q1c, ba1, bb1, bc1, eps1.reshape(1, 1),
        c1_b1.reshape(1, -1), c1_W2, c1_b2.reshape(1, -1),
        cov.reshape(1, -1), head_W1, head_b1.reshape(1, -1),
        head_W2.reshape(1, -1), head_b2.reshape(1, -1))
    return out


# speedup vs baseline: 4.1374x; 1.3797x over previous
import functools

import jax
import jax.numpy as jnp
from jax import lax
from jax.experimental import pallas as pl
from jax.experimental.pallas import tpu as pltpu
from jax.experimental.pallas import tpu_sc as plsc

V = 100000
N = 100000
E = 1600000
HID = 48
EMB = 24
L = 16
NCH = 3
NC = 2
NS = 16
NW = NC * NS

BN = 1000
GT = N // BN

EB = 128
NB_FULL = N // EB
TAIL_N = N - NB_FULL * EB
BPT = (NB_FULL + 1 + NW - 1) // NW

EPT = E // NW
NB_E = EPT // EB
TAIL_E = EPT - NB_E * EB
ROWS_T = 6256
ROWS_LAST = N - (NS - 1) * ROWS_T


def _tc_pre_body(emb_ref, xa_ref, xd_ref, wab_ref, bab_ref, wdist_ref,
                 bdist_ref, w1a_ref, w1t_ref, w1d_ref,
                 ta_ref, tb_ref, tc_ref, ba_ref, bb_ref, bc_ref):
    e = emb_ref[...]
    T = jnp.dot(e, w1t_ref[...], preferred_element_type=jnp.float32)
    ab = xa_ref[...]
    dist = xd_ref[...]
    a_emb = jax.nn.relu(ab * wab_ref[...] + bab_ref[...])
    d_emb = jax.nn.relu(dist * wdist_ref[...] + bdist_ref[...])
    base = (jnp.dot(a_emb, w1a_ref[...], preferred_element_type=jnp.float32)
            + jnp.dot(d_emb, w1d_ref[...], preferred_element_type=jnp.float32))
    ta_ref[...] = T[:, 0:L]
    tb_ref[...] = T[:, L:2 * L]
    tc_ref[...] = T[:, 2 * L:3 * L]
    ba_ref[...] = base[:, 0:L]
    bb_ref[...] = base[:, L:2 * L]
    bc_ref[...] = base[:, 2 * L:3 * L]


_tc_pre = pl.pallas_call(
    _tc_pre_body,
    grid=(GT,),
    in_specs=[
        pl.BlockSpec((BN, EMB), lambda i: (i, 0)),
        pl.BlockSpec((BN, 1), lambda i: (i, 0)),
        pl.BlockSpec((BN, 1), lambda i: (i, 0)),
        pl.BlockSpec((1, HID), lambda i: (0, 0)),
        pl.BlockSpec((1, HID), lambda i: (0, 0)),
        pl.BlockSpec((1, 32), lambda i: (0, 0)),
        pl.BlockSpec((1, 32), lambda i: (0, 0)),
        pl.BlockSpec((HID, HID), lambda i: (0, 0)),
        pl.BlockSpec((EMB, HID), lambda i: (0, 0)),
        pl.BlockSpec((32, HID), lambda i: (0, 0)),
    ],
    out_specs=[pl.BlockSpec((BN, L), lambda i: (i, 0))] * 6,
    out_shape=[jax.ShapeDtypeStruct((N, L), jnp.float32)] * 6,
)


def _sc_embed_body(tid_ref, ta_ref, tb_ref, tc_ref, b0_ref, b1_ref, b2_ref,
                   qa_ref, qb_ref, qc_ref,
                   idxf, trowf, browf, idxt, trowt, browt, sem):
    cid = lax.axis_index("c")
    sid = lax.axis_index("s")
    wid = cid * NS + sid
    Ts = (ta_ref, tb_ref, tc_ref)
    Bs = (b0_ref, b1_ref, b2_ref)
    Qs = (qa_ref, qb_ref, qc_ref)

    @pl.loop(0, BPT)
    def _batches(k):
        g = wid * BPT + k

        @pl.when(g < NB_FULL)
        def _full():
            r0 = g * EB
            pltpu.sync_copy(tid_ref.at[pl.ds(r0, EB)], idxf)
            for c in range(NCH):
                pltpu.async_copy(Ts[c].at[idxf], trowf, sem).wait()
                pltpu.sync_copy(Bs[c].at[pl.ds(r0, EB)], browf)
                for j in range(EB):
                    trowf[j, :] = trowf[j, :] + browf[j, :]
                pltpu.sync_copy(trowf, Qs[c].at[pl.ds(r0, EB)])

        @pl.when(g == NB_FULL)
        def _tail():
            r0 = NB_FULL * EB
            pltpu.sync_copy(tid_ref.at[pl.ds(r0, TAIL_N)], idxt)
            for c in range(NCH):
                pltpu.async_copy(Ts[c].at[idxt], trowt, sem).wait()
                pltpu.sync_copy(Bs[c].at[pl.ds(r0, TAIL_N)], browt)
                for j in range(TAIL_N):
                    trowt[j, :] = trowt[j, :] + browt[j, :]
                pltpu.sync_copy(trowt, Qs[c].at[pl.ds(r0, TAIL_N)])


@functools.cache
def _get_sc_embed():
    return pl.kernel(
        _sc_embed_body,
        out_type=[jax.ShapeDtypeStruct((N, L), jnp.float32)] * 3,
        mesh=plsc.VectorSubcoreMesh(core_axis_name="c", subcore_axis_name="s"),
        compiler_params=pltpu.CompilerParams(use_tc_tiling_on_sc=False),
        scratch_types=[
            pltpu.VMEM((EB,), jnp.int32),
            pltpu.VMEM((EB, L), jnp.float32),
            pltpu.VMEM((EB, L), jnp.float32),
            pltpu.VMEM((TAIL_N,), jnp.int32),
            pltpu.VMEM((TAIL_N, L), jnp.float32),
            pltpu.VMEM((TAIL_N, L), jnp.float32),
            pltpu.SemaphoreType.DMA,
        ],
    )


def _sc_scatter_body(qa_ref, qb_ref, qc_ref, src_ref, dst_ref, zero_ref,
                     aa_ref, ab_ref, ac_ref,
                     acc, srcf, dstf, rowf, srcg, dstg, rowg,
                     srct, dstt, rowt, sem, sem2):
    cid = lax.axis_index("c")
    sid = lax.axis_index("s")
    Qs = (qa_ref, qb_ref, qc_ref)
    As = (aa_ref, ab_ref, ac_ref)
    e_base = (cid * NS + sid) * EPT
    e_last = e_base + (NB_E - 1) * EB
    r0 = sid * ROWS_T

    for c in range(NCH):
        @pl.when(sid < NS - 1)
        def _zero():
            pltpu.sync_copy(zero_ref.at[pl.ds(0, ROWS_T)],
                            acc.at[pl.ds(r0, ROWS_T)])

        @pl.when(sid == NS - 1)
        def _zero_last():
            pltpu.sync_copy(zero_ref.at[pl.ds(0, ROWS_LAST)],
                            acc.at[pl.ds(r0, ROWS_LAST)])

        plsc.subcore_barrier()

        pltpu.sync_copy(src_ref.at[pl.ds(e_base, EB)], srcf)
        pltpu.sync_copy(dst_ref.at[pl.ds(e_base, EB)], dstf)

        @pl.loop(0, NB_E, step=2)
        def _edge_batch(k):
            e1 = e_base + (k + 1) * EB
            e2 = jnp.minimum(e_base + (k + 2) * EB, e_last)
            g1 = pltpu.async_copy(Qs[c].at[srcf], rowf, sem)
            pltpu.sync_copy(src_ref.at[pl.ds(e1, EB)], srcg)
            pltpu.sync_copy(dst_ref.at[pl.ds(e1, EB)], dstg)
            g1.wait()
            g2 = pltpu.async_copy(Qs[c].at[srcg], rowg, sem2)
            pltpu.sync_copy(rowf, acc.at[dstf], add=True)
            pltpu.sync_copy(src_ref.at[pl.ds(e2, EB)], srcf)
            pltpu.sync_copy(dst_ref.at[pl.ds(e2, EB)], dstf)
            g2.wait()
            pltpu.sync_copy(rowg, acc.at[dstg], add=True)

        et = e_base + NB_E * EB
        pltpu.sync_copy(src_ref.at[pl.ds(et, TAIL_E)], srct)
        pltpu.sync_copy(dst_ref.at[pl.ds(et, TAIL_E)], dstt)
        pltpu.async_copy(Qs[c].at[srct], rowt, sem).wait()
        pltpu.sync_copy(rowt, acc.at[dstt], add=True)

        plsc.subcore_barrier()

        o0 = cid * N + r0

        @pl.when(sid < NS - 1)
        def _dump():
            pltpu.sync_copy(acc.at[pl.ds(r0, ROWS_T)],
                            As[c].at[pl.ds(o0, ROWS_T)])

        @pl.when(sid == NS - 1)
        def _dump_last():
            pltpu.sync_copy(acc.at[pl.ds(r0, ROWS_LAST)],
                            As[c].at[pl.ds(o0, ROWS_LAST)])

        plsc.subcore_barrier()


@functools.cache
def _get_sc_scatter():
    return pl.kernel(
        _sc_scatter_body,
        out_type=[jax.ShapeDtypeStruct((NC * N, L), jnp.float32)] * 3,
        mesh=plsc.VectorSubcoreMesh(core_axis_name="c", subcore_axis_name="s"),
        compiler_params=pltpu.CompilerParams(use_tc_tiling_on_sc=False),
        scratch_types=[
            pltpu.VMEM_SHARED((N, L), jnp.float32),
            pltpu.VMEM((EB,), jnp.int32),
            pltpu.VMEM((EB,), jnp.int32),
            pltpu.VMEM((EB, L), jnp.float32),
            pltpu.VMEM((EB,), jnp.int32),
            pltpu.VMEM((EB,), jnp.int32),
            pltpu.VMEM((EB, L), jnp.float32),
            pltpu.VMEM((TAIL_E,), jnp.int32),
            pltpu.VMEM((TAIL_E,), jnp.int32),
            pltpu.VMEM((TAIL_E, L), jnp.float32),
            pltpu.SemaphoreType.DMA,
            pltpu.SemaphoreType.DMA,
        ],
    )


def _tc_mlp1_body(qa_ref, qb_ref, qc_ref, aa_ref, ab_ref, ac_ref,
                  eps_ref, b1_ref, w2_ref, b2_ref, wn_ref,
                  oa_ref, ob_ref, oc_ref):
    q = jnp.concatenate([qa_ref[...], qb_ref[...], qc_ref[...]], axis=1)
    aa = aa_ref[...]
    ab = ab_ref[...]
    ac = ac_ref[...]
    ag = jnp.concatenate([aa[0] + aa[1], ab[0] + ab[1], ac[0] + ac[1]],
                         axis=1)
    u = jax.nn.relu((1.0 + eps_ref[0, 0]) * q + ag + b1_ref[...])
    h = jax.nn.relu(
        jnp.dot(u, w2_ref[...], preferred_element_type=jnp.float32)
        + b2_ref[...])
    qn = jnp.dot(h, wn_ref[...], preferred_element_type=jnp.float32)
    oa_ref[...] = qn[:, 0:L]
    ob_ref[...] = qn[:, L:2 * L]
    oc_ref[...] = qn[:, 2 * L:3 * L]


_tc_mlp1 = pl.pallas_call(
    _tc_mlp1_body,
    grid=(GT,),
    in_specs=[
        pl.BlockSpec((BN, L), lambda i: (i, 0)),
        pl.BlockSpec((BN, L), lambda i: (i, 0)),
        pl.BlockSpec((BN, L), lambda i: (i, 0)),
        pl.BlockSpec((NC, BN, L), lambda i: (0, i, 0)),
        pl.BlockSpec((NC, BN, L), lambda i: (0, i, 0)),
        pl.BlockSpec((NC, BN, L), lambda i: (0, i, 0)),
        pl.BlockSpec((1, 1), lambda i: (0, 0)),
        pl.BlockSpec((1, HID), lambda i: (0, 0)),
        pl.BlockSpec((HID, HID), lambda i: (0, 0)),
        pl.BlockSpec((1, HID), lambda i: (0, 0)),
        pl.BlockSpec((HID, HID), lambda i: (0, 0)),
    ],
    out_specs=[pl.BlockSpec((BN, L), lambda i: (i, 0))] * 3,
    out_shape=[jax.ShapeDtypeStruct((N, L), jnp.float32)] * 3,
)


def _tc_mlp2_body(qa_ref, qb_ref, qc_ref, aa_ref, ab_ref, ac_ref,
                  eps_ref, b1_ref, w2_ref, b2_ref,
                  cov_ref, hw1_ref, hb1_ref, hw2_ref, hb2_ref,
                  out_ref, acc_ref):
    i = pl.program_id(0)
    q = jnp.concatenate([qa_ref[...], qb_ref[...], qc_ref[...]], axis=1)
    aa = aa_ref[...]
    ab = ab_ref[...]
    ac = ac_ref[...]
    ag = jnp.concatenate([aa[0] + aa[1], ab[0] + ab[1], ac[0] + ac[1]],
                         axis=1)
    u = jax.nn.relu((1.0 + eps_ref[0, 0]) * q + ag + b1_ref[...])
    h = jax.nn.relu(
        jnp.dot(u, w2_ref[...], preferred_element_type=jnp.float32)
        + b2_ref[...])
    bsum = jnp.sum(h, axis=0, keepdims=True)

    @pl.when(i == 0)
    def _init():
        acc_ref[...] = bsum

    @pl.when(i > 0)
    def _accum():
        acc_ref[...] = acc_ref[...] + bsum

    @pl.when(i == GT - 1)
    def _head():
        gm = acc_ref[...] * (1.0 / N)
        comb = jnp.concatenate([gm, cov_ref[...]], axis=1)
        t = jax.nn.relu(
            jnp.dot(comb.astype(jnp.bfloat16),
                    hw1_ref[...].astype(jnp.bfloat16),
                    preferred_element_type=jnp.float32)
            + hb1_ref[...])
        out_ref[...] = (
            jnp.sum(t * hw2_ref[...], axis=1, keepdims=True)
            + hb2_ref[...])


_tc_mlp2 = pl.pallas_call(
    _tc_mlp2_body,
    grid=(GT,),
    in_specs=[
        pl.BlockSpec((BN, L), lambda i: (i, 0)),
        pl.BlockSpec((BN, L), lambda i: (i, 0)),
        pl.BlockSpec((BN, L), lambda i: (i, 0)),
        pl.BlockSpec((NC, BN, L), lambda i: (0, i, 0)),
        pl.BlockSpec((NC, BN, L), lambda i: (0, i, 0)),
        pl.BlockSpec((NC, BN, L), lambda i: (0, i, 0)),
        pl.BlockSpec((1, 1), lambda i: (0, 0)),
        pl.BlockSpec((1, HID), lambda i: (0, 0)),
        pl.BlockSpec((HID, HID), lambda i: (0, 0)),
        pl.BlockSpec((1, HID), lambda i: (0, 0)),
        pl.BlockSpec((1, 4), lambda i: (0, 0)),
        pl.BlockSpec((HID + 4, 64), lambda i: (0, 0)),
        pl.BlockSpec((1, 64), lambda i: (0, 0)),
        pl.BlockSpec((1, 64), lambda i: (0, 0)),
        pl.BlockSpec((1, 1), lambda i: (0, 0)),
    ],
    out_specs=pl.BlockSpec((1, 1), lambda i: (0, 0)),
    out_shape=jax.ShapeDtypeStruct((1, 1), jnp.float32),
    scratch_shapes=[pltpu.VMEM((1, HID), jnp.float32)],
)


def kernel(x, edge_index, cov, W_ab, b_ab, emb, W_dist, b_dist, eps0,
           c0_W1, c0_b1, c0_W2, c0_b2, eps1, c1_W1, c1_b1, c1_W2, c1_b2,
           head_W1, head_b1, head_W2, head_b2):
    tid = jnp.clip(x[:, 1].astype(jnp.int32), 0, V - 1)
    xa = x[:, 0:1]
    xd = x[:, 2:3]
    src = edge_index[0]
    dst = edge_index[1]
    zeros = jnp.zeros((ROWS_T, L), jnp.float32)

    w1a = c0_W1[0:HID]
    w1t = c0_W1[HID:HID + EMB]
    w1d = c0_W1[HID + EMB:]

    ta, tb, tc, ba, bb, bc = _tc_pre(
        emb, xa, xd, W_ab, b_ab.reshape(1, -1), W_dist,
        b_dist.reshape(1, -1), w1a, w1t, w1d)

    qa, qb, qc = _get_sc_embed()(tid, ta, tb, tc, ba, bb, bc)

    aa, ab, ac = _get_sc_scatter()(qa, qb, qc, src, dst, zeros)
    aa = aa.reshape(NC, N, L)
    ab = ab.reshape(NC, N, L)
    ac = ac.reshape(NC, N, L)

    q1a, q1b, q1c = _tc_mlp1(
        qa, qb, qc, aa, ab, ac, eps0.reshape(1, 1), c0_b1.reshape(1, -1),
        c0_W2, c0_b2.reshape(1, -1), c1_W1)

    ba1, bb1, bc1 = _get_sc_scatter()(q1a, q1b, q1c, src, dst, zeros)
    ba1 = ba1.reshape(NC, N, L)
    bb1 = bb1.reshape(NC, N, L)
    bc1 = bc1.reshape(NC, N, L)

    out = _tc_mlp2(
        q1a, q1b, q1c, ba1, bb1, bc1, eps1.reshape(1, 1),
        c1_b1.reshape(1, -1), c1_W2, c1_b2.reshape(1, -1),
        cov.reshape(1, -1), head_W1, head_b1.reshape(1, -1),
        head_W2.reshape(1, -1), head_b2.reshape(1, -1))
    return out
